# baseline (speedup 1.0000x reference)
import functools

import jax
import jax.numpy as jnp
from jax import lax
from jax.experimental import pallas as pl
from jax.experimental.pallas import tpu as pltpu
from jax.experimental.pallas import tpu_sc as plsc

_N = 4096
_T = 200
_EMB = 64
_NC = 2
_NS = 16
_NW = _NC * _NS
_SPW = _N // _NW
_NBUF = 4
_CB = _T - 128


def _split_tc(toks):
    def body(x_ref, o0_ref, o1_ref):
        x = x_ref[...]
        o0_ref[...] = x[:, 0:128]
        o1_ref[...] = x[:, _T - 128:_T]

    return pl.pallas_call(
        body,
        out_shape=(
            jax.ShapeDtypeStruct((_N, 128), jnp.int32),
            jax.ShapeDtypeStruct((_N, 128), jnp.int32),
        ),
    )(toks)


def _pooled_sc(flat0, flat1, tok_emb):
    mesh = plsc.VectorSubcoreMesh(core_axis_name="c", subcore_axis_name="s")

    @functools.partial(
        pl.kernel,
        out_type=jax.ShapeDtypeStruct((_N, _EMB), jnp.float32),
        mesh=mesh,
        scratch_types=[
            pltpu.VMEM((_SPW, 128), jnp.int32),
            pltpu.VMEM((_SPW, 128), jnp.int32),
            [pltpu.VMEM((128 if b % 2 == 0 else _CB, _EMB), jnp.float32)
             for b in range(_NBUF)],
            pltpu.VMEM((_SPW, _EMB), jnp.float32),
            [pltpu.SemaphoreType.DMA for _ in range(_NBUF)],
        ],
        compiler_params=pltpu.CompilerParams(use_tc_tiling_on_sc=False),
    )
    def k(f0_hbm, f1_hbm, table_hbm, out_hbm, idx0, idx1, rows, pool_v, sems):
        wid = lax.axis_index("s") * _NC + lax.axis_index("c")
        base = wid * _SPW
        pltpu.sync_copy(f0_hbm.at[pl.ds(base, _SPW)], idx0)
        pltpu.sync_copy(f1_hbm.at[pl.ds(base, _SPW)], idx1)

        def start(q, slot, odd):
            s = q // 2
            if odd:
                pltpu.async_copy(
                    table_hbm.at[idx1.at[s, pl.ds(128 - _CB, _CB)]],
                    rows[slot], sems[slot],
                )
            else:
                pltpu.async_copy(
                    table_hbm.at[idx0.at[s]], rows[slot], sems[slot]
                )

        def wait(q, slot, odd):
            s = q // 2
            if odd:
                pltpu.make_async_copy(
                    table_hbm.at[idx1.at[s, pl.ds(128 - _CB, _CB)]],
                    rows[slot], sems[slot],
                ).wait()
            else:
                pltpu.make_async_copy(
                    table_hbm.at[idx0.at[s]], rows[slot], sems[slot]
                ).wait()

        def accum(ref, cnt, accs):
            def body(i, accs):
                for dt in range(4):
                    t = i * 4 + dt
                    accs = tuple(
                        a + ref[t, pl.ds(16 * cc, 16)] for cc, a in enumerate(accs)
                    )
                return accs
            return lax.fori_loop(0, cnt // 4, body, accs)

        inv_t = jnp.float32(1.0 / _T)
        zero4 = (jnp.zeros((16,), jnp.float32),) * 4
        qmask = 2 * _SPW - 1

        start(0, 0, False)
        start(1, 1, True)
        start(2, 2, False)

        def pair_body(p, carry):
            q0 = 4 * p
            for half in range(2):
                s = 2 * p + half
                accs = zero4
                for j2 in range(2):
                    j = 2 * half + j2
                    slot = j
                    nq = (q0 + j + _NBUF - 1) & qmask
                    start(nq, (j + _NBUF - 1) % _NBUF, (j + _NBUF - 1) % 2 == 1)
                    wait(q0 + j, slot, j2 == 1)
                    accs = accum(rows[slot], 128 if j2 == 0 else _CB, accs)
                for cc, acc in enumerate(accs):
                    pool_v[s, pl.ds(16 * cc, 16)] = acc * inv_t
            return carry

        lax.fori_loop(0, _SPW // 2, pair_body, 0)
        wait(0, 0, False)
        wait(1, 1, True)
        wait(2, 2, False)
        pltpu.sync_copy(pool_v, out_hbm.at[pl.ds(base, _SPW)])

    return k(flat0, flat1, tok_emb)


def _head_tc(pooled, W, b2, stdev11, noise):
    def body(p_ref, w_ref, b_ref, s_ref, n_ref, o_ref):
        p = p_ref[...]
        w = w_ref[...]
        logits = lax.dot_general(
            p, w, (((1,), (0,)), ((), ())), preferred_element_type=jnp.float32
        )
        x = logits + b_ref[...] + n_ref[...] * s_ref[0, 0]
        m = jnp.max(x, axis=-1, keepdims=True)
        e = jnp.exp(x - m)
        o_ref[...] = (x - m) - jnp.log(jnp.sum(e, axis=-1, keepdims=True))

    return pl.pallas_call(
        body,
        out_shape=jax.ShapeDtypeStruct((_N, 2), jnp.float32),
    )(pooled, W, b2, stdev11, noise)


def kernel(toks, tok_emb, W, b, stdev):
    flat0, flat1 = _split_tc(toks)
    pooled = _pooled_sc(flat0, flat1, tok_emb)
    noise = jax.random.normal(jax.random.key(1234), (_N, 2), dtype=jnp.float32)
    return _head_tc(pooled, W, b.reshape(1, 2), stdev.reshape(1, 1), noise)


# speedup vs baseline: 1.0314x; 1.0314x over previous
import functools

import jax
import jax.numpy as jnp
from jax import lax
from jax.experimental import pallas as pl
from jax.experimental.pallas import tpu as pltpu
from jax.experimental.pallas import tpu_sc as plsc

_N = 4096
_T = 200
_EMB = 64
_NC = 2
_NS = 16
_NW = _NC * _NS
_SPW = _N // _NW
_NBUF = 8
_CB = _T - 128


def _split_tc(toks):
    def body(x_ref, o0_ref, o1_ref):
        x = x_ref[...]
        o0_ref[...] = x[:, 0:128]
        o1_ref[...] = x[:, _T - 128:_T]

    return pl.pallas_call(
        body,
        out_shape=(
            jax.ShapeDtypeStruct((_N, 128), jnp.int32),
            jax.ShapeDtypeStruct((_N, 128), jnp.int32),
        ),
    )(toks)


def _pooled_sc(flat0, flat1, tok_emb):
    mesh = plsc.VectorSubcoreMesh(core_axis_name="c", subcore_axis_name="s")

    @functools.partial(
        pl.kernel,
        out_type=jax.ShapeDtypeStruct((_N, _EMB), jnp.float32),
        mesh=mesh,
        scratch_types=[
            pltpu.VMEM((_SPW, 128), jnp.int32),
            pltpu.VMEM((_SPW, 128), jnp.int32),
            [pltpu.VMEM((128 if b % 2 == 0 else _CB, _EMB), jnp.float32)
             for b in range(_NBUF)],
            pltpu.VMEM((_SPW, _EMB), jnp.float32),
            [pltpu.SemaphoreType.DMA for _ in range(_NBUF)],
        ],
        compiler_params=pltpu.CompilerParams(use_tc_tiling_on_sc=False),
    )
    def k(f0_hbm, f1_hbm, table_hbm, out_hbm, idx0, idx1, rows, pool_v, sems):
        wid = lax.axis_index("s") * _NC + lax.axis_index("c")
        base = wid * _SPW
        pltpu.sync_copy(f0_hbm.at[pl.ds(base, _SPW)], idx0)
        pltpu.sync_copy(f1_hbm.at[pl.ds(base, _SPW)], idx1)

        def start(q, slot, odd):
            s = q // 2
            if odd:
                pltpu.async_copy(
                    table_hbm.at[idx1.at[s, pl.ds(128 - _CB, _CB)]],
                    rows[slot], sems[slot],
                )
            else:
                pltpu.async_copy(
                    table_hbm.at[idx0.at[s]], rows[slot], sems[slot]
                )

        def wait(q, slot, odd):
            s = q // 2
            if odd:
                pltpu.make_async_copy(
                    table_hbm.at[idx1.at[s, pl.ds(128 - _CB, _CB)]],
                    rows[slot], sems[slot],
                ).wait()
            else:
                pltpu.make_async_copy(
                    table_hbm.at[idx0.at[s]], rows[slot], sems[slot]
                ).wait()

        def accum(ref, cnt, accs):
            def body(i, accs):
                for dt in range(8):
                    t = i * 8 + dt
                    accs = tuple(
                        a + ref[t, pl.ds(16 * cc, 16)] for cc, a in enumerate(accs)
                    )
                return accs
            return lax.fori_loop(0, cnt // 8, body, accs)

        inv_t = jnp.float32(1.0 / _T)
        zero4 = (jnp.zeros((16,), jnp.float32),) * 4
        qmask = 2 * _SPW - 1

        for c in range(_NBUF - 1):
            start(c, c, c % 2 == 1)

        def quad_body(p, carry):
            q0 = _NBUF * p
            for half in range(_NBUF // 2):
                s = (_NBUF // 2) * p + half
                accs = zero4
                for j2 in range(2):
                    j = 2 * half + j2
                    nq = (q0 + j + _NBUF - 1) & qmask
                    start(nq, (j + _NBUF - 1) % _NBUF, (j + _NBUF - 1) % 2 == 1)
                    wait(q0 + j, j, j2 == 1)
                    accs = accum(rows[j], 128 if j2 == 0 else _CB, accs)
                for cc, acc in enumerate(accs):
                    pool_v[s, pl.ds(16 * cc, 16)] = acc * inv_t
            return carry

        lax.fori_loop(0, 2 * _SPW // _NBUF, quad_body, 0)
        for c in range(_NBUF - 1):
            wait(c, c, c % 2 == 1)
        pltpu.sync_copy(pool_v, out_hbm.at[pl.ds(base, _SPW)])

    return k(flat0, flat1, tok_emb)


def _head_tc(pooled, W, b2, stdev11, noise):
    def body(p_ref, w_ref, b_ref, s_ref, n_ref, o_ref):
        p = p_ref[...]
        w = w_ref[...]
        logits = lax.dot_general(
            p, w, (((1,), (0,)), ((), ())), preferred_element_type=jnp.float32
        )
        x = logits + b_ref[...] + n_ref[...] * s_ref[0, 0]
        m = jnp.max(x, axis=-1, keepdims=True)
        e = jnp.exp(x - m)
        o_ref[...] = (x - m) - jnp.log(jnp.sum(e, axis=-1, keepdims=True))

    return pl.pallas_call(
        body,
        out_shape=jax.ShapeDtypeStruct((_N, 2), jnp.float32),
    )(pooled, W, b2, stdev11, noise)


def kernel(toks, tok_emb, W, b, stdev):
    flat0, flat1 = _split_tc(toks)
    pooled = _pooled_sc(flat0, flat1, tok_emb)
    noise = jax.random.normal(jax.random.key(1234), (_N, 2), dtype=jnp.float32)
    return _head_tc(pooled, W, b.reshape(1, 2), stdev.reshape(1, 1), noise)
